# Initial kernel scaffold; baseline (speedup 1.0000x reference)
import jax
import jax.numpy as jnp
from jax.experimental import pallas as pl


def kernel(features, edges, l1_w0W, l1_w0b, l1_w1W, l1_w1b, l2_w0W, l2_w0b, l2_w1W, l2_w1b, l3_w0W, l3_w0b, l3_w1W, l3_w1b, lf_w0W, lf_w0b, lf_w1W, lf_w1b):
    raise NotImplementedError("write your pallas kernel here")


# speedup vs baseline: 1.8875x; 1.8875x over previous
import functools

import jax
import jax.numpy as jnp
from jax import lax
from jax.experimental import pallas as pl
from jax.experimental.pallas import tpu as pltpu
from jax.experimental.pallas import tpu_sc as plsc

_N = 10000
_NP = 10240
_NPS = _NP // 16
_NST = _NPS // 8
_W = 128
_E2 = 640000
_NW = 32
_K = 128
_C = 160
_CB = 32
_CG = _C // _CB
_PAD = _NW * _C * _K - _E2

_mesh = plsc.VectorSubcoreMesh(core_axis_name="c", subcore_axis_name="s")


@functools.partial(
    pl.kernel,
    out_type=jax.ShapeDtypeStruct((2, _NP, _W), jnp.float32),
    mesh=_mesh,
    scratch_types=[
        pltpu.VMEM_SHARED((_NP, _W), jnp.float32),
        pltpu.VMEM((_NST, _W), jnp.float32),
        pltpu.VMEM((_CB, _K), jnp.int32),
        pltpu.VMEM((_CB, _K), jnp.int32),
        pltpu.VMEM((_K, _W), jnp.float32),
        pltpu.SemaphoreType.DMA,
    ],
)
def _sc_scatter(xw1, srcm, dstm, zrows, out, nbr_s, stage, srcv, dstv, rows,
                sem):
  cid = lax.axis_index("c")
  sid = lax.axis_index("s")
  wid = sid * 2 + cid
  r0 = sid * _NPS
  pltpu.sync_copy(zrows, stage)

  def zero_body(t, carry):
    pltpu.sync_copy(stage, nbr_s.at[pl.ds(r0 + t * _NST, _NST)])
    return carry

  lax.fori_loop(0, 8, zero_body, 0)
  plsc.subcore_barrier()

  def group(g, carry):
    pltpu.sync_copy(srcm.at[wid, pl.ds(g * _CB, _CB)], srcv)
    pltpu.sync_copy(dstm.at[wid, pl.ds(g * _CB, _CB)], dstv)

    def chunk(jc, c2):
      pltpu.async_copy(xw1.at[srcv.at[jc]], rows, sem).wait()
      pltpu.sync_copy(rows, nbr_s.at[dstv.at[jc]], add=True)
      return c2

    lax.fori_loop(0, _CB, chunk, 0)
    return carry

  lax.fori_loop(0, _CG, group, 0)
  plsc.subcore_barrier()

  def wb_body(t, carry):
    pltpu.sync_copy(nbr_s.at[pl.ds(r0 + t * _NST, _NST)], stage)
    pltpu.sync_copy(stage, out.at[cid, pl.ds(r0 + t * _NST, _NST)])
    return carry

  lax.fori_loop(0, 8, wb_body, 0)


@functools.partial(
    pl.kernel,
    out_type=jax.ShapeDtypeStruct((_NW, _NP), jnp.float32),
    mesh=_mesh,
    compiler_params=pltpu.CompilerParams(needs_layout_passes=False),
    scratch_types=[
        pltpu.VMEM((_NP,), jnp.float32),
        pltpu.VMEM((_C * _K,), jnp.int32),
    ],
)
def _sc_counts(dflat, out, counts, dstv):
  cid = lax.axis_index("c")
  sid = lax.axis_index("s")
  wid = sid * 2 + cid
  zero16 = jnp.zeros((16,), jnp.float32)

  def zero_body(i, carry):
    counts[pl.ds(i * 16, 16)] = zero16
    return carry

  lax.fori_loop(0, _NP // 16, zero_body, 0)
  pltpu.sync_copy(dflat.at[wid], dstv)
  ones = jnp.ones((16,), jnp.float32)

  def count_body(t, carry):
    idx = dstv[pl.ds(t * 16, 16)]
    plsc.addupdate_scatter(counts, [idx], ones)
    return carry

  lax.fori_loop(0, (_C * _K) // 16, count_body, 0)
  pltpu.sync_copy(counts, out.at[wid])


def _tc_linear2(x, w0, b0, w1, b1):
  n = x.shape[0]
  d0, d1 = w0.shape[0], w1.shape[0]

  def body(x_ref, w0_ref, b0_ref, w1_ref, b1_ref, o0_ref, o1_ref):
    xv = x_ref[...]
    dn = (((1,), (1,)), ((), ()))
    o0_ref[...] = lax.dot_general(
        xv, w0_ref[...], dn, preferred_element_type=jnp.float32) + b0_ref[...]
    o1_ref[...] = lax.dot_general(
        xv, w1_ref[...], dn, preferred_element_type=jnp.float32) + b1_ref[...]

  return pl.pallas_call(
      body,
      out_shape=(jax.ShapeDtypeStruct((n, d0), jnp.float32),
                 jax.ShapeDtypeStruct((n, d1), jnp.float32)),
  )(x, w0, b0, w1, b1)


def _tc_combine1(cpt, xw0, nbr):
  n, w = xw0.shape

  def body(cp_ref, xw0_ref, nbr_ref, o_ref, dinv_ref):
    dinv = 1.0 / jnp.sum(cp_ref[...], axis=1, keepdims=True)
    dinv_ref[...] = dinv
    s = xw0_ref[...] + nbr_ref[0] + nbr_ref[1]
    o_ref[...] = jnp.maximum(dinv * s, 0.0)

  return pl.pallas_call(
      body,
      out_shape=(jax.ShapeDtypeStruct((n, w), jnp.float32),
                 jax.ShapeDtypeStruct((n, 1), jnp.float32)),
  )(cpt, xw0, nbr)


def _tc_combine(dinv, xw0, nbr, relu):
  n, w = xw0.shape

  def body(dinv_ref, xw0_ref, nbr_ref, o_ref):
    s = xw0_ref[...] + nbr_ref[0] + nbr_ref[1]
    o = dinv_ref[...] * s
    if relu:
      o = jnp.maximum(o, 0.0)
    o_ref[...] = o

  return pl.pallas_call(
      body,
      out_shape=jax.ShapeDtypeStruct((n, w), jnp.float32),
  )(dinv, xw0, nbr)


def _pad_w(w, b):
  d, din = w.shape
  w = jnp.pad(w, ((0, _W - d), (0, _W - din)))
  b = jnp.pad(b, (0, _W - d))
  return w, b.reshape(1, -1)


def kernel(features, edges,
           l1_w0W, l1_w0b, l1_w1W, l1_w1b,
           l2_w0W, l2_w0b, l2_w1W, l2_w1b,
           l3_w0W, l3_w0b, l3_w1W, l3_w1b,
           lf_w0W, lf_w0b, lf_w1W, lf_w1b):
  ei = edges[:, 0]
  ej = edges[:, 1]
  padi = jnp.full((_PAD,), _N, jnp.int32)
  src = jnp.concatenate([ej, ei, padi]).reshape(_NW, _C, _K)
  dst_flat = jnp.concatenate([ei, ej, padi])
  dstm = dst_flat.reshape(_NW, _C, _K)
  dflat = dst_flat.reshape(_NW, _C * _K)

  x = jnp.pad(features, ((0, _NP - _N), (0, 0)))
  cparts = _sc_counts(dflat)
  cpt = cparts.T
  zrows = jnp.zeros((_NST, _W), jnp.float32)
  dep = (cparts[0, 0] * 0.0).astype(jnp.int32)
  src1 = src + dep
  dstm1 = dstm + dep

  layers = [
      (l1_w0W, l1_w0b, l1_w1W, l1_w1b, True),
      (l2_w0W, l2_w0b, l2_w1W, l2_w1b, True),
      (l3_w0W, l3_w0b, l3_w1W, l3_w1b, True),
      (lf_w0W, lf_w0b, lf_w1W, lf_w1b, False),
  ]
  dinv = None
  for li, (w0, b0, w1, b1, relu) in enumerate(layers):
    w0p, b0p = _pad_w(w0, b0)
    w1p, b1p = _pad_w(w1, b1)
    xw0, xw1 = _tc_linear2(x, w0p, b0p, w1p, b1p)
    if li == 0:
      nbr = _sc_scatter(xw1, src1, dstm1, zrows)
    else:
      nbr = _sc_scatter(xw1, src, dstm, zrows)
    if li == 0:
      x, dinv = _tc_combine1(cpt, xw0, nbr)
    else:
      x = _tc_combine(dinv, xw0, nbr, relu)
  return x[:_N, :3]


# speedup vs baseline: 4.3688x; 2.3146x over previous
import functools

import jax
import jax.numpy as jnp
from jax import lax
from jax.experimental import pallas as pl
from jax.experimental.pallas import tpu as pltpu
from jax.experimental.pallas import tpu_sc as plsc

_N = 10000
_NP = 10240
_NPS = _NP // 16
_NST = _NPS // 8
_E2 = 640000
_NW = 32
_K = 128
_C = 160
_CB = 32
_CG = _C // _CB
_PAD = _NW * _C * _K - _E2

_mesh = plsc.VectorSubcoreMesh(core_axis_name="c", subcore_axis_name="s")


def _make_sc_scatter(w):
  @functools.partial(
      pl.kernel,
      out_type=jax.ShapeDtypeStruct((2, _NP, w), jnp.float32),
      mesh=_mesh,
      compiler_params=pltpu.CompilerParams(use_tc_tiling_on_sc=False),
      scratch_types=[
          pltpu.VMEM_SHARED((_NP, w), jnp.float32),
          pltpu.VMEM((_NST, w), jnp.float32),
          pltpu.VMEM((_CB, _K), jnp.int32),
          pltpu.VMEM((_CB, _K), jnp.int32),
          pltpu.VMEM((_K, w), jnp.float32),
          pltpu.SemaphoreType.DMA,
      ],
  )
  def sc_scatter(xw1, srcm, dstm, zrows, out, nbr_s, stage, srcv, dstv, rows,
                 sem):
    cid = lax.axis_index("c")
    sid = lax.axis_index("s")
    wid = sid * 2 + cid
    r0 = sid * _NPS
    pltpu.sync_copy(zrows, stage)

    def zero_body(t, carry):
      pltpu.sync_copy(stage, nbr_s.at[pl.ds(r0 + t * _NST, _NST)])
      return carry

    lax.fori_loop(0, 8, zero_body, 0)
    plsc.subcore_barrier()

    def group(g, carry):
      pltpu.sync_copy(srcm.at[wid, pl.ds(g * _CB, _CB)], srcv)
      pltpu.sync_copy(dstm.at[wid, pl.ds(g * _CB, _CB)], dstv)

      def chunk(jc, c2):
        pltpu.async_copy(xw1.at[srcv.at[jc]], rows, sem).wait()
        pltpu.sync_copy(rows, nbr_s.at[dstv.at[jc]], add=True)
        return c2

      lax.fori_loop(0, _CB, chunk, 0)
      return carry

    lax.fori_loop(0, _CG, group, 0)
    plsc.subcore_barrier()

    def wb_body(t, carry):
      pltpu.sync_copy(nbr_s.at[pl.ds(r0 + t * _NST, _NST)], stage)
      pltpu.sync_copy(stage, out.at[cid, pl.ds(r0 + t * _NST, _NST)])
      return carry

    lax.fori_loop(0, 8, wb_body, 0)

  return sc_scatter


_sc_scatter = {w: _make_sc_scatter(w) for w in (96, 64, 32, 16)}


@functools.partial(
    pl.kernel,
    out_type=jax.ShapeDtypeStruct((_NW, _NP), jnp.float32),
    mesh=_mesh,
    compiler_params=pltpu.CompilerParams(needs_layout_passes=False),
    scratch_types=[
        pltpu.VMEM((_NP,), jnp.float32),
        pltpu.VMEM((_C * _K,), jnp.int32),
    ],
)
def _sc_counts(dflat, out, counts, dstv):
  cid = lax.axis_index("c")
  sid = lax.axis_index("s")
  wid = sid * 2 + cid
  zero16 = jnp.zeros((16,), jnp.float32)

  def zero_body(i, carry):
    counts[pl.ds(i * 16, 16)] = zero16
    return carry

  lax.fori_loop(0, _NP // 16, zero_body, 0)
  pltpu.sync_copy(dflat.at[wid], dstv)
  ones = jnp.ones((16,), jnp.float32)

  def count_body(t, carry):
    idx = dstv[pl.ds(t * 16, 16)]
    plsc.addupdate_scatter(counts, [idx], ones)
    return carry

  lax.fori_loop(0, (_C * _K) // 16, count_body, 0)
  pltpu.sync_copy(counts, out.at[wid])


def _tc_linear2(x, w0, b0, w1, b1):
  n = x.shape[0]
  d0, d1 = w0.shape[0], w1.shape[0]

  def body(x_ref, w0_ref, b0_ref, w1_ref, b1_ref, o0_ref, o1_ref):
    xv = x_ref[...]
    dn = (((1,), (1,)), ((), ()))
    o0_ref[...] = lax.dot_general(
        xv, w0_ref[...], dn, preferred_element_type=jnp.float32) + b0_ref[...]
    o1_ref[...] = lax.dot_general(
        xv, w1_ref[...], dn, preferred_element_type=jnp.float32) + b1_ref[...]

  return pl.pallas_call(
      body,
      out_shape=(jax.ShapeDtypeStruct((n, d0), jnp.float32),
                 jax.ShapeDtypeStruct((n, d1), jnp.float32)),
  )(x, w0, b0, w1, b1)


def _tc_combine1(cpt, xw0, nbr):
  n, w = xw0.shape

  def body(cp_ref, xw0_ref, nbr_ref, o_ref, dinv_ref):
    dinv = 1.0 / jnp.sum(cp_ref[...], axis=1, keepdims=True)
    dinv_ref[...] = dinv
    s = xw0_ref[...] + nbr_ref[0] + nbr_ref[1]
    o_ref[...] = jnp.maximum(dinv * s, 0.0)

  return pl.pallas_call(
      body,
      out_shape=(jax.ShapeDtypeStruct((n, w), jnp.float32),
                 jax.ShapeDtypeStruct((n, 1), jnp.float32)),
  )(cpt, xw0, nbr)


def _tc_combine(dinv, xw0, nbr, relu):
  n, w = xw0.shape

  def body(dinv_ref, xw0_ref, nbr_ref, o_ref):
    s = xw0_ref[...] + nbr_ref[0] + nbr_ref[1]
    o = dinv_ref[...] * s
    if relu:
      o = jnp.maximum(o, 0.0)
    o_ref[...] = o

  return pl.pallas_call(
      body,
      out_shape=jax.ShapeDtypeStruct((n, w), jnp.float32),
  )(dinv, xw0, nbr)


def _pad_w(w, b, wp):
  d = w.shape[0]
  if d < wp:
    w = jnp.pad(w, ((0, wp - d), (0, 0)))
    b = jnp.pad(b, (0, wp - d))
  return w, b.reshape(1, -1)


def kernel(features, edges,
           l1_w0W, l1_w0b, l1_w1W, l1_w1b,
           l2_w0W, l2_w0b, l2_w1W, l2_w1b,
           l3_w0W, l3_w0b, l3_w1W, l3_w1b,
           lf_w0W, lf_w0b, lf_w1W, lf_w1b):
  ei = edges[:, 0]
  ej = edges[:, 1]
  padi = jnp.full((_PAD,), _N, jnp.int32)
  src = jnp.concatenate([ej, ei, padi]).reshape(_NW, _C, _K)
  dst_flat = jnp.concatenate([ei, ej, padi])
  dstm = dst_flat.reshape(_NW, _C, _K)
  dflat = dst_flat.reshape(_NW, _C * _K)

  x = jnp.pad(features, ((0, _NP - _N), (0, 0)))
  cparts = _sc_counts(dflat)
  cpt = cparts.T
  dep = (cparts[0, 0] * 0.0).astype(jnp.int32)
  src1 = src + dep
  dstm1 = dstm + dep

  layers = [
      (l1_w0W, l1_w0b, l1_w1W, l1_w1b, 96, True),
      (l2_w0W, l2_w0b, l2_w1W, l2_w1b, 64, True),
      (l3_w0W, l3_w0b, l3_w1W, l3_w1b, 32, True),
      (lf_w0W, lf_w0b, lf_w1W, lf_w1b, 16, False),
  ]
  dinv = None
  for li, (w0, b0, w1, b1, wp, relu) in enumerate(layers):
    w0p, b0p = _pad_w(w0, b0, wp)
    w1p, b1p = _pad_w(w1, b1, wp)
    xw0, xw1 = _tc_linear2(x, w0p, b0p, w1p, b1p)
    zrows = jnp.zeros((_NST, wp), jnp.float32)
    if li == 0:
      nbr = _sc_scatter[wp](xw1, src1, dstm1, zrows)
    else:
      nbr = _sc_scatter[wp](xw1, src, dstm, zrows)
    if li == 0:
      x, dinv = _tc_combine1(cpt, xw0, nbr)
    else:
      x = _tc_combine(dinv, xw0, nbr, relu)
  return x[:_N, :3]
